# baseline (speedup 1.0000x reference)
import jax
import jax.numpy as jnp
from jax import lax
from jax.experimental import pallas as pl
from jax.experimental.pallas import tpu as pltpu
from jax.experimental.pallas import tpu_sc as plsc

N = 100000
VOX = 729
VOXP = 736
D = 9
LW = 128
P = 2000
NB = N // P

NC, NS = 2, 16
NW = NC * NS
CH = 128
ROUNDS = 26
B_PER_W = ROUNDS * CH
NPAD = NW * B_PER_W


def _mesh():
    return plsc.VectorSubcoreMesh(core_axis_name="c", subcore_axis_name="s",
                                  num_cores=NC, num_subcores=NS)


def _sc_passb_body(g1t_hbm, cnc_r_hbm, cc_r_hbm, zeros_hbm, s2b_hbm,
                   acc, idxa, idxb, buf0, buf1, sem0, sem1):
    c = lax.axis_index("c")
    s = lax.axis_index("s")
    wid = s * NC + c

    @pl.when(s == 0)
    def _():
        pltpu.sync_copy(zeros_hbm, acc)
    pltpu.sync_copy(cnc_r_hbm.at[wid], idxa)
    pltpu.sync_copy(cc_r_hbm.at[wid], idxb)
    plsc.subcore_barrier()

    pltpu.async_copy(g1t_hbm.at[idxa.at[0]], buf0, sem0)
    pltpu.async_copy(g1t_hbm.at[idxa.at[1]], buf1, sem1)

    def body(j, carry):
        r0 = 2 * j
        pltpu.make_async_copy(g1t_hbm.at[idxa.at[r0]], buf0, sem0).wait()
        pltpu.sync_copy(buf0, acc.at[idxb.at[r0]], add=True)

        @pl.when(r0 + 2 < ROUNDS)
        def _():
            pltpu.async_copy(g1t_hbm.at[idxa.at[r0 + 2]], buf0, sem0)
        r1 = 2 * j + 1
        pltpu.make_async_copy(g1t_hbm.at[idxa.at[r1]], buf1, sem1).wait()
        pltpu.sync_copy(buf1, acc.at[idxb.at[r1]], add=True)

        @pl.when(r1 + 2 < ROUNDS)
        def _():
            pltpu.async_copy(g1t_hbm.at[idxa.at[r1 + 2]], buf1, sem1)
        return carry

    lax.fori_loop(0, ROUNDS // 2, body, 0)
    plsc.subcore_barrier()

    @pl.when(s == 0)
    def _():
        pltpu.sync_copy(acc, s2b_hbm.at[c])


def _sc_gather_body(gct_hbm, cnc_r_hbm, outer_hbm, idxa, buf0, buf1,
                    sem0, sem1):
    c = lax.axis_index("c")
    s = lax.axis_index("s")
    wid = s * NC + c
    base_w = wid * B_PER_W

    pltpu.sync_copy(cnc_r_hbm.at[wid], idxa)
    pltpu.async_copy(gct_hbm.at[idxa.at[0]], buf0, sem0)
    pltpu.async_copy(gct_hbm.at[idxa.at[1]], buf1, sem1)

    def body(j, carry):
        r0 = 2 * j
        pltpu.make_async_copy(gct_hbm.at[idxa.at[r0]], buf0, sem0).wait()
        pltpu.sync_copy(buf0, outer_hbm.at[pl.ds(base_w + r0 * CH, CH)])

        @pl.when(r0 + 2 < ROUNDS)
        def _():
            pltpu.async_copy(gct_hbm.at[idxa.at[r0 + 2]], buf0, sem0)
        r1 = 2 * j + 1
        pltpu.make_async_copy(gct_hbm.at[idxa.at[r1]], buf1, sem1).wait()
        pltpu.sync_copy(buf1, outer_hbm.at[pl.ds(base_w + r1 * CH, CH)])

        @pl.when(r1 + 2 < ROUNDS)
        def _():
            pltpu.async_copy(gct_hbm.at[idxa.at[r1 + 2]], buf1, sem1)
        return carry

    lax.fori_loop(0, ROUNDS // 2, body, 0)


def _make_sc_kernels():
    sc_passb = pl.kernel(
        _sc_passb_body,
        out_type=jax.ShapeDtypeStruct((NC, VOXP, LW), jnp.float32),
        mesh=_mesh(),
        scratch_types=[pltpu.VMEM_SHARED((VOXP, LW), jnp.float32),
                       pltpu.VMEM((ROUNDS, CH), jnp.int32),
                       pltpu.VMEM((ROUNDS, CH), jnp.int32),
                       pltpu.VMEM((CH, LW), jnp.float32),
                       pltpu.VMEM((CH, LW), jnp.float32),
                       pltpu.SemaphoreType.DMA,
                       pltpu.SemaphoreType.DMA],
    )
    sc_gather = pl.kernel(
        _sc_gather_body,
        out_type=jax.ShapeDtypeStruct((NPAD, LW), jnp.float32),
        mesh=_mesh(),
        scratch_types=[pltpu.VMEM((ROUNDS, CH), jnp.int32),
                       pltpu.VMEM((CH, LW), jnp.float32),
                       pltpu.VMEM((CH, LW), jnp.float32),
                       pltpu.SemaphoreType.DMA,
                       pltpu.SemaphoreType.DMA],
    )
    return sc_passb, sc_gather


def _onehot(idx, width):
    cols = jax.lax.broadcasted_iota(jnp.int32, (idx.shape[0], width), 1)
    return (idx[:, None] == cols).astype(jnp.float32)


def _hist_kernel(x_ref, cc_ref, cnc_ref, hcc_ref, hcnc_ref):
    @pl.when(pl.program_id(0) == 0)
    def _():
        hcc_ref[...] = jnp.zeros_like(hcc_ref)
        hcnc_ref[...] = jnp.zeros_like(hcnc_ref)

    x = x_ref[...]
    cc = cc_ref[0, 0, :]
    cnc = cnc_ref[0, 0, :]
    xb4 = jnp.concatenate([x, jnp.ones((P, 1), jnp.float32)], axis=1)
    oh_cc = _onehot(cc, VOX)
    oh_cnc = _onehot(cnc, VOX)
    hcc_ref[...] += jax.lax.dot_general(
        oh_cc, xb4, (((0,), (0,)), ((), ())),
        preferred_element_type=jnp.float32)
    hcnc_ref[...] += jax.lax.dot_general(
        oh_cnc, jnp.ones((P, 1), jnp.float32), (((0,), (0,)), ((), ())),
        preferred_element_type=jnp.float32)


def _rank_perm(pcnt):
    present = (pcnt > 0.0).astype(jnp.float32)
    rows = jax.lax.broadcasted_iota(jnp.int32, (VOX, VOX), 0)
    cols = jax.lax.broadcasted_iota(jnp.int32, (VOX, VOX), 1)
    lower = (cols < rows).astype(jnp.float32)
    rank = jax.lax.dot_general(lower, present, (((1,), (0,)), ((), ())),
                               preferred_element_type=jnp.float32)
    return (rank == cols.astype(jnp.float32)).astype(jnp.float32) * present


def _conv_grid(grid_vc, kr_ref, kb):
    cin = grid_vc.shape[1]
    cout = kr_ref.shape[2]
    pad = jnp.zeros((96, cin), jnp.float32)
    padded = jnp.concatenate([pad, grid_vc, pad], axis=0)
    v = jax.lax.broadcasted_iota(jnp.int32, (VOX, 1), 0)
    vz = v // 81
    vy = (v // 9) % 9
    vx = v % 9
    acc = jnp.zeros((VOX, cout), jnp.float32)
    for dz in range(3):
        for dy in range(3):
            for dx in range(3):
                o = (dz * 3 + dy) * 3 + dx
                k = (dz - 1) * 81 + (dy - 1) * 9 + (dx - 1)
                shifted = padded[96 + k: 96 + k + VOX, :]
                okz = jnp.logical_and(vz + (dz - 1) >= 0, vz + (dz - 1) < 9)
                oky = jnp.logical_and(vy + (dy - 1) >= 0, vy + (dy - 1) < 9)
                okx = jnp.logical_and(vx + (dx - 1) >= 0, vx + (dx - 1) < 9)
                m = jnp.logical_and(okz, jnp.logical_and(oky, okx))
                contrib = jnp.where(m, shifted, 0.0)
                acc += jax.lax.dot_general(
                    contrib, kr_ref[o], (((1,), (0,)), ((), ())),
                    preferred_element_type=jnp.float32)
    return acc + kb


def _pad_table(t64):
    return jnp.pad(t64, ((0, VOXP - VOX), (0, LW - 64)))


def _dense1_kernel(hcc_ref, hcnc_ref, w1_ref, b1_ref, k1r_ref, kb1_ref,
                   g1t_ref, s2a_ref):
    sums1 = hcc_ref[:, 0:3]
    cnt = hcc_ref[:, 3:4]
    pm = _rank_perm(hcnc_ref[...])
    means1 = sums1 / jnp.maximum(cnt, 1.0)
    grid1 = jax.lax.dot_general(pm, means1, (((1,), (0,)), ((), ())),
                                preferred_element_type=jnp.float32)
    g1t = _conv_grid(grid1, k1r_ref, kb1_ref[...])
    g1t_ref[...] = _pad_table(g1t)
    s2a_ref[...] = jax.lax.dot_general(
        sums1, w1_ref[...], (((1,), (1,)), ((), ())),
        preferred_element_type=jnp.float32) + cnt * b1_ref[...]


def _dense2_kernel(hcc_ref, hcnc_ref, s2a_ref, s2b_ref, g1t_ref,
                   w1_ref, b1_ref, w2_ref, b2_ref, k2r_ref, kb2_ref,
                   gct_ref, at_ref, bp_ref):
    cnt = hcc_ref[:, 3:4]
    pm = _rank_perm(hcnc_ref[...])
    sums2 = (s2a_ref[...] + s2b_ref[0, 0:VOX, 0:64]
             + s2b_ref[1, 0:VOX, 0:64])
    means2 = sums2 / jnp.maximum(cnt, 1.0)
    grid2 = jax.lax.dot_general(pm, means2, (((1,), (0,)), ((), ())),
                                preferred_element_type=jnp.float32)
    g2t = _conv_grid(grid2, k2r_ref, kb2_ref[...])
    gct = g2t + jax.lax.dot_general(
        g1t_ref[0:VOX, 0:64], w2_ref[...], (((1,), (1,)), ((), ())),
        preferred_element_type=jnp.float32)
    gct_ref[...] = _pad_table(gct)
    at_ref[...] = jax.lax.dot_general(
        w1_ref[...], w2_ref[...], (((0,), (1,)), ((), ())),
        preferred_element_type=jnp.float32)
    bp_ref[...] = jax.lax.dot_general(
        b1_ref[...], w2_ref[...], (((1,), (1,)), ((), ())),
        preferred_element_type=jnp.float32) + b2_ref[...]


def _out_kernel(x_ref, outer_ref, at_ref, bp_ref, out_ref):
    inner = jax.lax.dot_general(x_ref[...], at_ref[...],
                                (((1,), (0,)), ((), ())),
                                preferred_element_type=jnp.float32)
    out_ref[...] = inner + outer_ref[:, 0:64] + bp_ref[...]


@jax.jit
def kernel(x, consecutive_cluster, cluster_non_consecutive,
           W1, b1, K1, kb1, W2, b2, K2, kb2):
    f32 = jnp.float32
    cc = consecutive_cluster.astype(jnp.int32)
    cnc = cluster_non_consecutive.astype(jnp.int32)
    cc3 = cc.reshape(NB, 1, P)
    cnc3 = cnc.reshape(NB, 1, P)
    cc_pad = jnp.concatenate(
        [cc, jnp.zeros((NPAD - N,), jnp.int32)]).reshape(NW, ROUNDS, CH)
    cnc_pad = jnp.concatenate(
        [cnc, jnp.full((NPAD - N,), VOX, jnp.int32)]).reshape(NW, ROUNDS, CH)
    zeros_t = jnp.zeros((VOXP, LW), f32)
    k1r = jnp.transpose(K1, (2, 3, 4, 1, 0)).reshape(27, 3, 64)
    k2r = jnp.transpose(K2, (2, 3, 4, 1, 0)).reshape(27, 64, 64)
    b1r = b1.reshape(1, 64)
    b2r = b2.reshape(1, 64)
    kb1r = kb1.reshape(1, 64)
    kb2r = kb2.reshape(1, 64)
    sc_passb, sc_gather = _make_sc_kernels()

    idx_spec = pl.BlockSpec((1, 1, P), lambda i: (i, 0, 0))
    full = lambda s: pl.BlockSpec(s, lambda i: tuple(0 for _ in s))

    hcc, hcnc = pl.pallas_call(
        _hist_kernel,
        grid=(NB,),
        in_specs=[pl.BlockSpec((P, 3), lambda i: (i, 0)), idx_spec, idx_spec],
        out_specs=[full((VOX, 4)), full((VOX, 1))],
        out_shape=[jax.ShapeDtypeStruct((VOX, 4), f32),
                   jax.ShapeDtypeStruct((VOX, 1), f32)],
    )(x, cc3, cnc3)

    g1t, s2a = pl.pallas_call(
        _dense1_kernel,
        out_shape=[jax.ShapeDtypeStruct((VOXP, LW), f32),
                   jax.ShapeDtypeStruct((VOX, 64), f32)],
    )(hcc, hcnc, W1, b1r, k1r, kb1r)

    s2b = sc_passb(g1t, cnc_pad, cc_pad, zeros_t)

    gct, at, bp = pl.pallas_call(
        _dense2_kernel,
        out_shape=[jax.ShapeDtypeStruct((VOXP, LW), f32),
                   jax.ShapeDtypeStruct((3, 64), f32),
                   jax.ShapeDtypeStruct((1, 64), f32)],
    )(hcc, hcnc, s2a, s2b, g1t, W1, b1r, W2, b2r, k2r, kb2r)

    outer = sc_gather(gct, cnc_pad)

    out = pl.pallas_call(
        _out_kernel,
        grid=(NB,),
        in_specs=[pl.BlockSpec((P, 3), lambda i: (i, 0)),
                  pl.BlockSpec((P, LW), lambda i: (i, 0)),
                  pl.BlockSpec((3, 64), lambda i: (0, 0)),
                  pl.BlockSpec((1, 64), lambda i: (0, 0))],
        out_specs=pl.BlockSpec((P, 64), lambda i: (i, 0)),
        out_shape=jax.ShapeDtypeStruct((N, 64), f32),
    )(x, outer[0:N], at, bp)
    return out


# speedup vs baseline: 1.8753x; 1.8753x over previous
import jax
import jax.numpy as jnp
from jax import lax
from jax.experimental import pallas as pl
from jax.experimental.pallas import tpu as pltpu
from jax.experimental.pallas import tpu_sc as plsc

N = 100000
VOX = 729
VOXP = 736
D = 9
LW = 128
P = 2000
NB = N // P

NC, NS = 2, 16
NW = NC * NS
CH = 128
ROUNDS = 25
B_PER_W = ROUNDS * CH
NPAD = NW * B_PER_W


def _mesh():
    return plsc.VectorSubcoreMesh(core_axis_name="c", subcore_axis_name="s",
                                  num_cores=NC, num_subcores=NS)


def _sc_passb_body(g1t_hbm, cnc_hbm, cc_hbm, zeros_hbm, s2b_hbm,
                   acc, idx_v, idx2_v, rows_v, sem):
    c = lax.axis_index("c")
    s = lax.axis_index("s")
    wid = s * NC + c

    @pl.when(s == 0)
    def _():
        pltpu.sync_copy(zeros_hbm, acc)
    plsc.subcore_barrier()

    def round(i, carry):
        base = wid * B_PER_W + i * CH
        pltpu.sync_copy(cnc_hbm.at[pl.ds(base, CH)], idx_v)
        pltpu.async_copy(g1t_hbm.at[idx_v], rows_v, sem).wait()
        pltpu.sync_copy(cc_hbm.at[pl.ds(base, CH)], idx2_v)
        pltpu.sync_copy(rows_v, acc.at[idx2_v], add=True)
        return carry

    lax.fori_loop(0, ROUNDS, round, 0)
    plsc.subcore_barrier()

    @pl.when(s == 0)
    def _():
        pltpu.sync_copy(acc, s2b_hbm.at[c])


def _make_sc_kernels():
    sc_passb = pl.kernel(
        _sc_passb_body,
        out_type=jax.ShapeDtypeStruct((NC, VOXP, LW), jnp.float32),
        mesh=_mesh(),
        scratch_types=[pltpu.VMEM_SHARED((VOXP, LW), jnp.float32),
                       pltpu.VMEM((CH,), jnp.int32),
                       pltpu.VMEM((CH,), jnp.int32),
                       pltpu.VMEM((CH, LW), jnp.float32),
                       pltpu.SemaphoreType.DMA],
    )
    return sc_passb


def _onehot(idx, width):
    cols = jax.lax.broadcasted_iota(jnp.int32, (idx.shape[0], width), 1)
    return (idx[:, None] == cols).astype(jnp.float32)


def _hist_kernel(x_ref, cc_ref, cnc_ref, hcc_ref, hcnc_ref):
    @pl.when(pl.program_id(0) == 0)
    def _():
        hcc_ref[...] = jnp.zeros_like(hcc_ref)
        hcnc_ref[...] = jnp.zeros_like(hcnc_ref)

    x = x_ref[...]
    cc = cc_ref[0, 0, :]
    cnc = cnc_ref[0, 0, :]
    xb4 = jnp.concatenate([x, jnp.ones((P, 1), jnp.float32)], axis=1)
    oh_cc = _onehot(cc, VOX)
    oh_cnc = _onehot(cnc, VOX)
    hcc_ref[...] += jax.lax.dot_general(
        oh_cc, xb4, (((0,), (0,)), ((), ())),
        preferred_element_type=jnp.float32)
    hcnc_ref[...] += jax.lax.dot_general(
        oh_cnc, jnp.ones((P, 1), jnp.float32), (((0,), (0,)), ((), ())),
        preferred_element_type=jnp.float32)


def _rank_perm(pcnt):
    present = (pcnt > 0.0).astype(jnp.float32)
    rows = jax.lax.broadcasted_iota(jnp.int32, (VOX, VOX), 0)
    cols = jax.lax.broadcasted_iota(jnp.int32, (VOX, VOX), 1)
    lower = (cols < rows).astype(jnp.float32)
    rank = jax.lax.dot_general(lower, present, (((1,), (0,)), ((), ())),
                               preferred_element_type=jnp.float32)
    return (rank == cols.astype(jnp.float32)).astype(jnp.float32) * present


def _conv_grid(grid_vc, kr_ref, kb):
    cin = grid_vc.shape[1]
    cout = kr_ref.shape[2]
    pad = jnp.zeros((96, cin), jnp.float32)
    padded = jnp.concatenate([pad, grid_vc, pad], axis=0)
    v = jax.lax.broadcasted_iota(jnp.int32, (VOX, 1), 0)
    vz = v // 81
    vy = (v // 9) % 9
    vx = v % 9
    acc = jnp.zeros((VOX, cout), jnp.float32)
    for dz in range(3):
        for dy in range(3):
            for dx in range(3):
                o = (dz * 3 + dy) * 3 + dx
                k = (dz - 1) * 81 + (dy - 1) * 9 + (dx - 1)
                shifted = padded[96 + k: 96 + k + VOX, :]
                okz = jnp.logical_and(vz + (dz - 1) >= 0, vz + (dz - 1) < 9)
                oky = jnp.logical_and(vy + (dy - 1) >= 0, vy + (dy - 1) < 9)
                okx = jnp.logical_and(vx + (dx - 1) >= 0, vx + (dx - 1) < 9)
                m = jnp.logical_and(okz, jnp.logical_and(oky, okx))
                contrib = jnp.where(m, shifted, 0.0)
                acc += jax.lax.dot_general(
                    contrib, kr_ref[o], (((1,), (0,)), ((), ())),
                    preferred_element_type=jnp.float32)
    return acc + kb


def _pad_table(t64):
    return jnp.pad(t64, ((0, VOXP - VOX), (0, LW - 64)))


def _dense1_kernel(hcc_ref, hcnc_ref, w1_ref, b1_ref, k1r_ref, kb1_ref,
                   g1t_ref, s2a_ref):
    sums1 = hcc_ref[:, 0:3]
    cnt = hcc_ref[:, 3:4]
    pm = _rank_perm(hcnc_ref[...])
    means1 = sums1 / jnp.maximum(cnt, 1.0)
    grid1 = jax.lax.dot_general(pm, means1, (((1,), (0,)), ((), ())),
                                preferred_element_type=jnp.float32)
    g1t = _conv_grid(grid1, k1r_ref, kb1_ref[...])
    g1t_ref[...] = _pad_table(g1t)
    s2a_ref[...] = jax.lax.dot_general(
        sums1, w1_ref[...], (((1,), (1,)), ((), ())),
        preferred_element_type=jnp.float32) + cnt * b1_ref[...]


def _dense2_kernel(hcc_ref, hcnc_ref, s2a_ref, s2b_ref, g1t_ref,
                   w1_ref, b1_ref, w2_ref, b2_ref, k2r_ref, kb2_ref,
                   gct_ref, at_ref, bp_ref):
    cnt = hcc_ref[:, 3:4]
    pm = _rank_perm(hcnc_ref[...])
    sums2 = (s2a_ref[...] + s2b_ref[0, 0:VOX, 0:64]
             + s2b_ref[1, 0:VOX, 0:64])
    means2 = sums2 / jnp.maximum(cnt, 1.0)
    grid2 = jax.lax.dot_general(pm, means2, (((1,), (0,)), ((), ())),
                                preferred_element_type=jnp.float32)
    g2t = _conv_grid(grid2, k2r_ref, kb2_ref[...])
    gct = g2t + jax.lax.dot_general(
        g1t_ref[0:VOX, 0:64], w2_ref[...], (((1,), (1,)), ((), ())),
        preferred_element_type=jnp.float32)
    gct_ref[...] = gct
    at_ref[...] = jax.lax.dot_general(
        w1_ref[...], w2_ref[...], (((0,), (1,)), ((), ())),
        preferred_element_type=jnp.float32)
    bp_ref[...] = jax.lax.dot_general(
        b1_ref[...], w2_ref[...], (((1,), (1,)), ((), ())),
        preferred_element_type=jnp.float32) + b2_ref[...]


def _out_kernel(x_ref, cnc_ref, gct_ref, at_ref, bp_ref, out_ref):
    cnc = cnc_ref[0, 0, :]
    oh_cnc = _onehot(cnc, VOX)
    outer = jax.lax.dot_general(oh_cnc, gct_ref[...], (((1,), (0,)), ((), ())),
                                preferred_element_type=jnp.float32)
    inner = jax.lax.dot_general(x_ref[...], at_ref[...],
                                (((1,), (0,)), ((), ())),
                                preferred_element_type=jnp.float32)
    out_ref[...] = inner + outer + bp_ref[...]


@jax.jit
def kernel(x, consecutive_cluster, cluster_non_consecutive,
           W1, b1, K1, kb1, W2, b2, K2, kb2):
    f32 = jnp.float32
    cc = consecutive_cluster.astype(jnp.int32)
    cnc = cluster_non_consecutive.astype(jnp.int32)
    cc3 = cc.reshape(NB, 1, P)
    cnc3 = cnc.reshape(NB, 1, P)
    cc_pad = jnp.concatenate([cc, jnp.zeros((NPAD - N,), jnp.int32)])
    cnc_pad = jnp.concatenate([cnc, jnp.full((NPAD - N,), VOX, jnp.int32)])
    zeros_t = jnp.zeros((VOXP, LW), f32)
    k1r = jnp.transpose(K1, (2, 3, 4, 1, 0)).reshape(27, 3, 64)
    k2r = jnp.transpose(K2, (2, 3, 4, 1, 0)).reshape(27, 64, 64)
    b1r = b1.reshape(1, 64)
    b2r = b2.reshape(1, 64)
    kb1r = kb1.reshape(1, 64)
    kb2r = kb2.reshape(1, 64)
    sc_passb = _make_sc_kernels()

    idx_spec = pl.BlockSpec((1, 1, P), lambda i: (i, 0, 0))
    full = lambda s: pl.BlockSpec(s, lambda i: tuple(0 for _ in s))

    hcc, hcnc = pl.pallas_call(
        _hist_kernel,
        grid=(NB,),
        in_specs=[pl.BlockSpec((P, 3), lambda i: (i, 0)), idx_spec, idx_spec],
        out_specs=[full((VOX, 4)), full((VOX, 1))],
        out_shape=[jax.ShapeDtypeStruct((VOX, 4), f32),
                   jax.ShapeDtypeStruct((VOX, 1), f32)],
    )(x, cc3, cnc3)

    g1t, s2a = pl.pallas_call(
        _dense1_kernel,
        out_shape=[jax.ShapeDtypeStruct((VOXP, LW), f32),
                   jax.ShapeDtypeStruct((VOX, 64), f32)],
    )(hcc, hcnc, W1, b1r, k1r, kb1r)

    s2b = sc_passb(g1t, cnc_pad, cc_pad, zeros_t)

    gct, at, bp = pl.pallas_call(
        _dense2_kernel,
        out_shape=[jax.ShapeDtypeStruct((VOX, 64), f32),
                   jax.ShapeDtypeStruct((3, 64), f32),
                   jax.ShapeDtypeStruct((1, 64), f32)],
    )(hcc, hcnc, s2a, s2b, g1t, W1, b1r, W2, b2r, k2r, kb2r)

    out = pl.pallas_call(
        _out_kernel,
        grid=(NB,),
        in_specs=[pl.BlockSpec((P, 3), lambda i: (i, 0)), idx_spec,
                  pl.BlockSpec((VOX, 64), lambda i: (0, 0)),
                  pl.BlockSpec((3, 64), lambda i: (0, 0)),
                  pl.BlockSpec((1, 64), lambda i: (0, 0))],
        out_specs=pl.BlockSpec((P, 64), lambda i: (i, 0)),
        out_shape=jax.ShapeDtypeStruct((N, 64), f32),
    )(x, cnc3, gct, at, bp)
    return out


# speedup vs baseline: 2.0275x; 1.0812x over previous
import jax
import jax.numpy as jnp
from jax import lax
from jax.experimental import pallas as pl
from jax.experimental.pallas import tpu as pltpu
from jax.experimental.pallas import tpu_sc as plsc

N = 100000
VOX = 729
VOXP = 736
D = 9
LW = 128
P = 2000
NB = N // P

NC, NS = 2, 16
NW = NC * NS
CH = 128
ROUNDS = 25
B_PER_W = ROUNDS * CH
NPAD = NW * B_PER_W


def _mesh():
    return plsc.VectorSubcoreMesh(core_axis_name="c", subcore_axis_name="s",
                                  num_cores=NC, num_subcores=NS)


def _sc_passb_body(g1t_hbm, cnc_hbm, cc_hbm, zeros_hbm, s2b_hbm,
                   acc, idxg0, idxg1, idxs, buf0, buf1, sem0, sem1):
    c = lax.axis_index("c")
    s = lax.axis_index("s")
    wid = s * NC + c
    base_w = wid * B_PER_W

    @pl.when(s == 0)
    def _():
        pltpu.sync_copy(zeros_hbm, acc)
    plsc.subcore_barrier()

    pltpu.sync_copy(cnc_hbm.at[pl.ds(base_w, CH)], idxg0)
    pltpu.async_copy(g1t_hbm.at[idxg0], buf0, sem0)

    def body(j, carry):
        r0 = 2 * j
        pltpu.sync_copy(cnc_hbm.at[pl.ds(base_w + (r0 + 1) * CH, CH)], idxg1)
        pltpu.async_copy(g1t_hbm.at[idxg1], buf1, sem1)
        pltpu.make_async_copy(g1t_hbm.at[idxg0], buf0, sem0).wait()
        pltpu.sync_copy(cc_hbm.at[pl.ds(base_w + r0 * CH, CH)], idxs)
        pltpu.sync_copy(buf0, acc.at[idxs], add=True)
        pltpu.sync_copy(cnc_hbm.at[pl.ds(base_w + (r0 + 2) * CH, CH)], idxg0)
        pltpu.async_copy(g1t_hbm.at[idxg0], buf0, sem0)
        pltpu.make_async_copy(g1t_hbm.at[idxg1], buf1, sem1).wait()
        pltpu.sync_copy(cc_hbm.at[pl.ds(base_w + (r0 + 1) * CH, CH)], idxs)
        pltpu.sync_copy(buf1, acc.at[idxs], add=True)
        return carry

    lax.fori_loop(0, (ROUNDS - 1) // 2, body, 0)
    pltpu.make_async_copy(g1t_hbm.at[idxg0], buf0, sem0).wait()
    pltpu.sync_copy(cc_hbm.at[pl.ds(base_w + (ROUNDS - 1) * CH, CH)], idxs)
    pltpu.sync_copy(buf0, acc.at[idxs], add=True)
    plsc.subcore_barrier()

    @pl.when(s == 0)
    def _():
        pltpu.sync_copy(acc, s2b_hbm.at[c])


def _make_sc_kernels():
    sc_passb = pl.kernel(
        _sc_passb_body,
        out_type=jax.ShapeDtypeStruct((NC, VOXP, LW), jnp.float32),
        mesh=_mesh(),
        scratch_types=[pltpu.VMEM_SHARED((VOXP, LW), jnp.float32),
                       pltpu.VMEM((CH,), jnp.int32),
                       pltpu.VMEM((CH,), jnp.int32),
                       pltpu.VMEM((CH,), jnp.int32),
                       pltpu.VMEM((CH, LW), jnp.float32),
                       pltpu.VMEM((CH, LW), jnp.float32),
                       pltpu.SemaphoreType.DMA,
                       pltpu.SemaphoreType.DMA],
    )
    return sc_passb


def _onehot(idx, width):
    cols = jax.lax.broadcasted_iota(jnp.int32, (idx.shape[0], width), 1)
    return (idx[:, None] == cols).astype(jnp.float32)


def _hist_kernel(x_ref, cc_ref, cnc_ref, hcc_ref, hcnc_ref):
    @pl.when(pl.program_id(0) == 0)
    def _():
        hcc_ref[...] = jnp.zeros_like(hcc_ref)
        hcnc_ref[...] = jnp.zeros_like(hcnc_ref)

    x = x_ref[...]
    cc = cc_ref[0, 0, :]
    cnc = cnc_ref[0, 0, :]
    xb4 = jnp.concatenate([x, jnp.ones((P, 1), jnp.float32)], axis=1)
    oh_cc = _onehot(cc, VOX)
    oh_cnc = _onehot(cnc, VOX)
    hcc_ref[...] += jax.lax.dot_general(
        oh_cc, xb4, (((0,), (0,)), ((), ())),
        preferred_element_type=jnp.float32)
    hcnc_ref[...] += jax.lax.dot_general(
        oh_cnc, jnp.ones((P, 1), jnp.float32), (((0,), (0,)), ((), ())),
        preferred_element_type=jnp.float32)


def _rank_perm(pcnt):
    present = (pcnt > 0.0).astype(jnp.float32)
    rows = jax.lax.broadcasted_iota(jnp.int32, (VOX, VOX), 0)
    cols = jax.lax.broadcasted_iota(jnp.int32, (VOX, VOX), 1)
    lower = (cols < rows).astype(jnp.float32)
    rank = jax.lax.dot_general(lower, present, (((1,), (0,)), ((), ())),
                               preferred_element_type=jnp.float32)
    return (rank == cols.astype(jnp.float32)).astype(jnp.float32) * present


def _conv_grid(grid_vc, kr_ref, kb):
    cin = grid_vc.shape[1]
    cout = kr_ref.shape[2]
    pad = jnp.zeros((96, cin), jnp.float32)
    padded = jnp.concatenate([pad, grid_vc, pad], axis=0)
    v = jax.lax.broadcasted_iota(jnp.int32, (VOX, 1), 0)
    vz = v // 81
    vy = (v // 9) % 9
    vx = v % 9
    acc = jnp.zeros((VOX, cout), jnp.float32)
    for dz in range(3):
        for dy in range(3):
            for dx in range(3):
                o = (dz * 3 + dy) * 3 + dx
                k = (dz - 1) * 81 + (dy - 1) * 9 + (dx - 1)
                shifted = padded[96 + k: 96 + k + VOX, :]
                okz = jnp.logical_and(vz + (dz - 1) >= 0, vz + (dz - 1) < 9)
                oky = jnp.logical_and(vy + (dy - 1) >= 0, vy + (dy - 1) < 9)
                okx = jnp.logical_and(vx + (dx - 1) >= 0, vx + (dx - 1) < 9)
                m = jnp.logical_and(okz, jnp.logical_and(oky, okx))
                contrib = jnp.where(m, shifted, 0.0)
                acc += jax.lax.dot_general(
                    contrib, kr_ref[o], (((1,), (0,)), ((), ())),
                    preferred_element_type=jnp.float32)
    return acc + kb


def _pad_table(t64):
    return jnp.pad(t64, ((0, VOXP - VOX), (0, LW - 64)))


def _dense1_kernel(hcc_ref, hcnc_ref, w1_ref, b1_ref, k1r_ref, kb1_ref,
                   g1t_ref, s2a_ref):
    sums1 = hcc_ref[:, 0:3]
    cnt = hcc_ref[:, 3:4]
    pm = _rank_perm(hcnc_ref[...])
    means1 = sums1 / jnp.maximum(cnt, 1.0)
    grid1 = jax.lax.dot_general(pm, means1, (((1,), (0,)), ((), ())),
                                preferred_element_type=jnp.float32)
    g1t = _conv_grid(grid1, k1r_ref, kb1_ref[...])
    g1t_ref[...] = _pad_table(g1t)
    s2a_ref[...] = jax.lax.dot_general(
        sums1, w1_ref[...], (((1,), (1,)), ((), ())),
        preferred_element_type=jnp.float32) + cnt * b1_ref[...]


def _dense2_kernel(hcc_ref, hcnc_ref, s2a_ref, s2b_ref, g1t_ref,
                   w1_ref, b1_ref, w2_ref, b2_ref, k2r_ref, kb2_ref,
                   gct_ref, at_ref, bp_ref):
    cnt = hcc_ref[:, 3:4]
    pm = _rank_perm(hcnc_ref[...])
    sums2 = (s2a_ref[...] + s2b_ref[0, 0:VOX, 0:64]
             + s2b_ref[1, 0:VOX, 0:64])
    means2 = sums2 / jnp.maximum(cnt, 1.0)
    grid2 = jax.lax.dot_general(pm, means2, (((1,), (0,)), ((), ())),
                                preferred_element_type=jnp.float32)
    g2t = _conv_grid(grid2, k2r_ref, kb2_ref[...])
    gct = g2t + jax.lax.dot_general(
        g1t_ref[0:VOX, 0:64], w2_ref[...], (((1,), (1,)), ((), ())),
        preferred_element_type=jnp.float32)
    gct_ref[...] = gct
    at_ref[...] = jax.lax.dot_general(
        w1_ref[...], w2_ref[...], (((0,), (1,)), ((), ())),
        preferred_element_type=jnp.float32)
    bp_ref[...] = jax.lax.dot_general(
        b1_ref[...], w2_ref[...], (((1,), (1,)), ((), ())),
        preferred_element_type=jnp.float32) + b2_ref[...]


def _out_kernel(x_ref, cnc_ref, gct_ref, at_ref, bp_ref, out_ref):
    cnc = cnc_ref[0, 0, :]
    oh_cnc = _onehot(cnc, VOX)
    outer = jax.lax.dot_general(oh_cnc, gct_ref[...], (((1,), (0,)), ((), ())),
                                preferred_element_type=jnp.float32)
    inner = jax.lax.dot_general(x_ref[...], at_ref[...],
                                (((1,), (0,)), ((), ())),
                                preferred_element_type=jnp.float32)
    out_ref[...] = inner + outer + bp_ref[...]


@jax.jit
def kernel(x, consecutive_cluster, cluster_non_consecutive,
           W1, b1, K1, kb1, W2, b2, K2, kb2):
    f32 = jnp.float32
    cc = consecutive_cluster.astype(jnp.int32)
    cnc = cluster_non_consecutive.astype(jnp.int32)
    cc3 = cc.reshape(NB, 1, P)
    cnc3 = cnc.reshape(NB, 1, P)
    cc_pad = jnp.concatenate([cc, jnp.zeros((NPAD - N,), jnp.int32)])
    cnc_pad = jnp.concatenate([cnc, jnp.full((NPAD - N,), VOX, jnp.int32)])
    zeros_t = jnp.zeros((VOXP, LW), f32)
    k1r = jnp.transpose(K1, (2, 3, 4, 1, 0)).reshape(27, 3, 64)
    k2r = jnp.transpose(K2, (2, 3, 4, 1, 0)).reshape(27, 64, 64)
    b1r = b1.reshape(1, 64)
    b2r = b2.reshape(1, 64)
    kb1r = kb1.reshape(1, 64)
    kb2r = kb2.reshape(1, 64)
    sc_passb = _make_sc_kernels()

    idx_spec = pl.BlockSpec((1, 1, P), lambda i: (i, 0, 0))
    full = lambda s: pl.BlockSpec(s, lambda i: tuple(0 for _ in s))

    hcc, hcnc = pl.pallas_call(
        _hist_kernel,
        grid=(NB,),
        in_specs=[pl.BlockSpec((P, 3), lambda i: (i, 0)), idx_spec, idx_spec],
        out_specs=[full((VOX, 4)), full((VOX, 1))],
        out_shape=[jax.ShapeDtypeStruct((VOX, 4), f32),
                   jax.ShapeDtypeStruct((VOX, 1), f32)],
    )(x, cc3, cnc3)

    g1t, s2a = pl.pallas_call(
        _dense1_kernel,
        out_shape=[jax.ShapeDtypeStruct((VOXP, LW), f32),
                   jax.ShapeDtypeStruct((VOX, 64), f32)],
    )(hcc, hcnc, W1, b1r, k1r, kb1r)

    s2b = sc_passb(g1t, cnc_pad, cc_pad, zeros_t)

    gct, at, bp = pl.pallas_call(
        _dense2_kernel,
        out_shape=[jax.ShapeDtypeStruct((VOX, 64), f32),
                   jax.ShapeDtypeStruct((3, 64), f32),
                   jax.ShapeDtypeStruct((1, 64), f32)],
    )(hcc, hcnc, s2a, s2b, g1t, W1, b1r, W2, b2r, k2r, kb2r)

    out = pl.pallas_call(
        _out_kernel,
        grid=(NB,),
        in_specs=[pl.BlockSpec((P, 3), lambda i: (i, 0)), idx_spec,
                  pl.BlockSpec((VOX, 64), lambda i: (0, 0)),
                  pl.BlockSpec((3, 64), lambda i: (0, 0)),
                  pl.BlockSpec((1, 64), lambda i: (0, 0))],
        out_specs=pl.BlockSpec((P, 64), lambda i: (i, 0)),
        out_shape=jax.ShapeDtypeStruct((N, 64), f32),
    )(x, cnc3, gct, at, bp)
    return out


# speedup vs baseline: 2.0279x; 1.0002x over previous
import jax
import jax.numpy as jnp
from jax import lax
from jax.experimental import pallas as pl
from jax.experimental.pallas import tpu as pltpu
from jax.experimental.pallas import tpu_sc as plsc

N = 100000
VOX = 729
VOXP = 736
D = 9
LW = 128
P = 2000
NB = N // P

NC, NS = 2, 16
NW = NC * NS
CH = 128
ROUNDS = 25
B_PER_W = ROUNDS * CH
NPAD = NW * B_PER_W


def _mesh():
    return plsc.VectorSubcoreMesh(core_axis_name="c", subcore_axis_name="s",
                                  num_cores=NC, num_subcores=NS)


def _sc_passb_body(g1t_hbm, cnc_hbm, cc_hbm, zeros_hbm, s2b_hbm,
                   acc, idxg0, idxg1, idxs, buf0, buf1, sem0, sem1):
    c = lax.axis_index("c")
    s = lax.axis_index("s")
    wid = s * NC + c
    base_w = wid * B_PER_W

    @pl.when(s == 0)
    def _():
        pltpu.sync_copy(zeros_hbm, acc)
    plsc.subcore_barrier()

    pltpu.sync_copy(cnc_hbm.at[pl.ds(base_w, CH)], idxg0)
    pltpu.async_copy(g1t_hbm.at[idxg0], buf0, sem0)

    def body(j, carry):
        r0 = 2 * j
        pltpu.sync_copy(cnc_hbm.at[pl.ds(base_w + (r0 + 1) * CH, CH)], idxg1)
        pltpu.async_copy(g1t_hbm.at[idxg1], buf1, sem1)
        pltpu.make_async_copy(g1t_hbm.at[idxg0], buf0, sem0).wait()
        pltpu.sync_copy(cc_hbm.at[pl.ds(base_w + r0 * CH, CH)], idxs)
        pltpu.sync_copy(buf0, acc.at[idxs], add=True)
        pltpu.sync_copy(cnc_hbm.at[pl.ds(base_w + (r0 + 2) * CH, CH)], idxg0)
        pltpu.async_copy(g1t_hbm.at[idxg0], buf0, sem0)
        pltpu.make_async_copy(g1t_hbm.at[idxg1], buf1, sem1).wait()
        pltpu.sync_copy(cc_hbm.at[pl.ds(base_w + (r0 + 1) * CH, CH)], idxs)
        pltpu.sync_copy(buf1, acc.at[idxs], add=True)
        return carry

    lax.fori_loop(0, (ROUNDS - 1) // 2, body, 0)
    pltpu.make_async_copy(g1t_hbm.at[idxg0], buf0, sem0).wait()
    pltpu.sync_copy(cc_hbm.at[pl.ds(base_w + (ROUNDS - 1) * CH, CH)], idxs)
    pltpu.sync_copy(buf0, acc.at[idxs], add=True)
    plsc.subcore_barrier()

    @pl.when(s == 0)
    def _():
        pltpu.sync_copy(acc, s2b_hbm.at[c])


def _make_sc_kernels():
    sc_passb = pl.kernel(
        _sc_passb_body,
        out_type=jax.ShapeDtypeStruct((NC, VOXP, LW), jnp.float32),
        mesh=_mesh(),
        scratch_types=[pltpu.VMEM_SHARED((VOXP, LW), jnp.float32),
                       pltpu.VMEM((CH,), jnp.int32),
                       pltpu.VMEM((CH,), jnp.int32),
                       pltpu.VMEM((CH,), jnp.int32),
                       pltpu.VMEM((CH, LW), jnp.float32),
                       pltpu.VMEM((CH, LW), jnp.float32),
                       pltpu.SemaphoreType.DMA,
                       pltpu.SemaphoreType.DMA],
    )
    return sc_passb


def _onehot(idx, width):
    cols = jax.lax.broadcasted_iota(jnp.int32, (idx.shape[0], width), 1)
    return (idx[:, None] == cols).astype(jnp.float32)


def _hist_kernel(x_ref, cc_ref, cnc_ref, hcc_ref, hcnc_ref):
    @pl.when(pl.program_id(0) == 0)
    def _():
        hcc_ref[...] = jnp.zeros_like(hcc_ref)
        hcnc_ref[...] = jnp.zeros_like(hcnc_ref)

    x = x_ref[...]
    cc = cc_ref[0, 0, :]
    cnc = cnc_ref[0, 0, :]
    xb4 = jnp.concatenate([x, jnp.ones((P, 1), jnp.float32)], axis=1)
    oh_cc = _onehot(cc, VOX)
    oh_cnc = _onehot(cnc, VOX)
    hcc_ref[...] += jax.lax.dot_general(
        oh_cc, xb4, (((0,), (0,)), ((), ())),
        preferred_element_type=jnp.float32)
    hcnc_ref[...] += jax.lax.dot_general(
        oh_cnc, jnp.ones((P, 1), jnp.float32), (((0,), (0,)), ((), ())),
        preferred_element_type=jnp.float32)


def _rank_perm(pcnt):
    present = (pcnt > 0.0).astype(jnp.float32)
    rows = jax.lax.broadcasted_iota(jnp.int32, (VOX, VOX), 0)
    cols = jax.lax.broadcasted_iota(jnp.int32, (VOX, VOX), 1)
    lower = (cols < rows).astype(jnp.float32)
    rank = jax.lax.dot_general(lower, present, (((1,), (0,)), ((), ())),
                               preferred_element_type=jnp.float32)
    return (rank == cols.astype(jnp.float32)).astype(jnp.float32) * present


def _conv_grid(grid_vc, kr_ref, kb):
    cin = grid_vc.shape[1]
    cout = kr_ref.shape[2]
    pad = jnp.zeros((96, cin), jnp.float32)
    padded = jnp.concatenate([pad, grid_vc, pad], axis=0)
    v = jax.lax.broadcasted_iota(jnp.int32, (VOX, 1), 0)
    vz = v // 81
    vy = (v // 9) % 9
    vx = v % 9
    acc = jnp.zeros((VOX, cout), jnp.float32)
    for dz in range(3):
        for dy in range(3):
            for dx in range(3):
                o = (dz * 3 + dy) * 3 + dx
                k = (dz - 1) * 81 + (dy - 1) * 9 + (dx - 1)
                shifted = padded[96 + k: 96 + k + VOX, :]
                okz = jnp.logical_and(vz + (dz - 1) >= 0, vz + (dz - 1) < 9)
                oky = jnp.logical_and(vy + (dy - 1) >= 0, vy + (dy - 1) < 9)
                okx = jnp.logical_and(vx + (dx - 1) >= 0, vx + (dx - 1) < 9)
                m = jnp.logical_and(okz, jnp.logical_and(oky, okx))
                contrib = jnp.where(m, shifted, 0.0)
                acc += jax.lax.dot_general(
                    contrib, kr_ref[o], (((1,), (0,)), ((), ())),
                    preferred_element_type=jnp.float32)
    return acc + kb


def _pad_table(t64):
    return jnp.pad(t64, ((0, VOXP - VOX), (0, LW - 64)))


def _dense1_kernel(hcc_ref, hcnc_ref, w1_ref, b1_ref, k1r_ref, kb1_ref,
                   g1t_ref, s2a_ref):
    sums1 = hcc_ref[:, 0:3]
    cnt = hcc_ref[:, 3:4]
    pm = _rank_perm(hcnc_ref[...])
    means1 = sums1 / jnp.maximum(cnt, 1.0)
    grid1 = jax.lax.dot_general(pm, means1, (((1,), (0,)), ((), ())),
                                preferred_element_type=jnp.float32)
    g1t = _conv_grid(grid1, k1r_ref, kb1_ref[...])
    g1t_ref[...] = _pad_table(g1t)
    s2a_ref[...] = jax.lax.dot_general(
        sums1, w1_ref[...], (((1,), (1,)), ((), ())),
        preferred_element_type=jnp.float32) + cnt * b1_ref[...]


def _dense2_kernel(hcc_ref, hcnc_ref, s2a_ref, s2b_ref, g1t_ref,
                   w1_ref, b1_ref, w2_ref, b2_ref, k2r_ref, kb2_ref,
                   gct_ref, at_ref, bp_ref):
    cnt = hcc_ref[:, 3:4]
    pm = _rank_perm(hcnc_ref[...])
    sums2 = (s2a_ref[...] + s2b_ref[0, 0:VOX, 0:64]
             + s2b_ref[1, 0:VOX, 0:64])
    means2 = sums2 / jnp.maximum(cnt, 1.0)
    grid2 = jax.lax.dot_general(pm, means2, (((1,), (0,)), ((), ())),
                                preferred_element_type=jnp.float32)
    g2t = _conv_grid(grid2, k2r_ref, kb2_ref[...])
    gct = g2t + jax.lax.dot_general(
        g1t_ref[0:VOX, 0:64], w2_ref[...], (((1,), (1,)), ((), ())),
        preferred_element_type=jnp.float32)
    gct_ref[...] = gct
    at_ref[...] = jax.lax.dot_general(
        w1_ref[...], w2_ref[...], (((0,), (1,)), ((), ())),
        preferred_element_type=jnp.float32)
    bp_ref[...] = jax.lax.dot_general(
        b1_ref[...], w2_ref[...], (((1,), (1,)), ((), ())),
        preferred_element_type=jnp.float32) + b2_ref[...]


def _out_kernel(x_ref, cnc_ref, gct_ref, at_ref, bp_ref, out_ref):
    cnc = cnc_ref[0, 0, :]
    cols = jax.lax.broadcasted_iota(jnp.int32, (P, VOX), 1)
    oh_cnc = (cnc[:, None] == cols).astype(jnp.bfloat16)
    outer = jax.lax.dot_general(oh_cnc, gct_ref[...].astype(jnp.bfloat16),
                                (((1,), (0,)), ((), ())),
                                preferred_element_type=jnp.float32)
    inner = jax.lax.dot_general(x_ref[...], at_ref[...],
                                (((1,), (0,)), ((), ())),
                                preferred_element_type=jnp.float32)
    out_ref[...] = inner + outer + bp_ref[...]


@jax.jit
def kernel(x, consecutive_cluster, cluster_non_consecutive,
           W1, b1, K1, kb1, W2, b2, K2, kb2):
    f32 = jnp.float32
    cc = consecutive_cluster.astype(jnp.int32)
    cnc = cluster_non_consecutive.astype(jnp.int32)
    cc3 = cc.reshape(NB, 1, P)
    cnc3 = cnc.reshape(NB, 1, P)
    cc_pad = jnp.concatenate([cc, jnp.zeros((NPAD - N,), jnp.int32)])
    cnc_pad = jnp.concatenate([cnc, jnp.full((NPAD - N,), VOX, jnp.int32)])
    zeros_t = jnp.zeros((VOXP, LW), f32)
    k1r = jnp.transpose(K1, (2, 3, 4, 1, 0)).reshape(27, 3, 64)
    k2r = jnp.transpose(K2, (2, 3, 4, 1, 0)).reshape(27, 64, 64)
    b1r = b1.reshape(1, 64)
    b2r = b2.reshape(1, 64)
    kb1r = kb1.reshape(1, 64)
    kb2r = kb2.reshape(1, 64)
    sc_passb = _make_sc_kernels()

    idx_spec = pl.BlockSpec((1, 1, P), lambda i: (i, 0, 0))
    full = lambda s: pl.BlockSpec(s, lambda i: tuple(0 for _ in s))

    hcc, hcnc = pl.pallas_call(
        _hist_kernel,
        grid=(NB,),
        in_specs=[pl.BlockSpec((P, 3), lambda i: (i, 0)), idx_spec, idx_spec],
        out_specs=[full((VOX, 4)), full((VOX, 1))],
        out_shape=[jax.ShapeDtypeStruct((VOX, 4), f32),
                   jax.ShapeDtypeStruct((VOX, 1), f32)],
    )(x, cc3, cnc3)

    g1t, s2a = pl.pallas_call(
        _dense1_kernel,
        out_shape=[jax.ShapeDtypeStruct((VOXP, LW), f32),
                   jax.ShapeDtypeStruct((VOX, 64), f32)],
    )(hcc, hcnc, W1, b1r, k1r, kb1r)

    s2b = sc_passb(g1t, cnc_pad, cc_pad, zeros_t)

    gct, at, bp = pl.pallas_call(
        _dense2_kernel,
        out_shape=[jax.ShapeDtypeStruct((VOX, 64), f32),
                   jax.ShapeDtypeStruct((3, 64), f32),
                   jax.ShapeDtypeStruct((1, 64), f32)],
    )(hcc, hcnc, s2a, s2b, g1t, W1, b1r, W2, b2r, k2r, kb2r)

    out = pl.pallas_call(
        _out_kernel,
        grid=(NB,),
        in_specs=[pl.BlockSpec((P, 3), lambda i: (i, 0)), idx_spec,
                  pl.BlockSpec((VOX, 64), lambda i: (0, 0)),
                  pl.BlockSpec((3, 64), lambda i: (0, 0)),
                  pl.BlockSpec((1, 64), lambda i: (0, 0))],
        out_specs=pl.BlockSpec((P, 64), lambda i: (i, 0)),
        out_shape=jax.ShapeDtypeStruct((N, 64), f32),
    )(x, cnc3, gct, at, bp)
    return out
